# Initial kernel scaffold; baseline (speedup 1.0000x reference)
import jax
import jax.numpy as jnp
from jax.experimental import pallas as pl


def kernel(x, edge_index, edge_attr, batch, params):
    raise NotImplementedError("write your pallas kernel here")


# speedup vs baseline: 7.8265x; 7.8265x over previous
import functools
import math

import jax
import jax.numpy as jnp
from jax import lax
from jax.experimental import pallas as pl
from jax.experimental.pallas import tpu as pltpu
from jax.experimental.pallas import tpu_sc as plsc

N = 50000
E = 800000
D_EDGE = 16
NHID = 64
NUM_SEEDS = 20
KPAD = 32

HALF = 25600
MROWS = 2 * HALF
TBL = 25920
DUMMY = 25600
NTILE = 16
STRIPE = TBL // NTILE
CH = 256
NSUB = CH // 128
NCHUNK = E // CH
CHC = 128
NSUBC = CHC // 128
NCHUNKC = E // CHC
OUTG = 64
BK = 2000
GRID = E // BK

_mesh = plsc.VectorSubcoreMesh(core_axis_name="c", subcore_axis_name="s")
_sc_params = pltpu.CompilerParams(use_tc_tiling_on_sc=False)


@functools.partial(
    pl.kernel,
    out_type=jax.ShapeDtypeStruct((MROWS, 16), jnp.float32),
    mesh=_mesh,
    compiler_params=_sc_params,
    scratch_types=[
        pltpu.VMEM((128, 16), jnp.float32),
        pltpu.VMEM((NSUB, 128), jnp.int32),
        pltpu.VMEM((NSUB, 128), jnp.int32),
        pltpu.VMEM((NSUB, 128), jnp.int32),
        pltpu.VMEM_SHARED((TBL, 16), jnp.float32),
    ],
)
def _deg_kernel(src_hbm, dst_hbm, deg_hbm, ones_v, sidx, didx, clamp2, table):
    c = lax.axis_index("c")
    s = lax.axis_index("s")
    half0 = c * HALF
    zvec = jnp.zeros((16,), jnp.float32)
    ovec = jnp.ones((16,), jnp.float32)

    def fill_ones(i, _):
        ones_v[i, :] = ovec
        return 0

    lax.fori_loop(0, 128, fill_ones, 0)

    def fill_zero(i, _):
        ones_v[i, :] = zvec
        return 0

    lax.fori_loop(0, 128, fill_zero, 0)

    def zero_tbl(t, _):
        pltpu.sync_copy(ones_v, table.at[pl.ds(s * STRIPE + t * 128, 128)])
        return 0

    lax.fori_loop(0, STRIPE // 128, zero_tbl, 0)
    pltpu.sync_copy(ones_v.at[pl.ds(0, STRIPE - (STRIPE // 128) * 128)],
                    table.at[pl.ds(s * STRIPE + (STRIPE // 128) * 128,
                                   STRIPE - (STRIPE // 128) * 128)])
    lax.fori_loop(0, 128, fill_ones, 0)
    plsc.subcore_barrier()

    def chunk_body(i, _):
        ch = s + NTILE * i

        @pl.when(ch < NCHUNK)
        def _():
            base = ch * CH
            for j in range(NSUB):
                pltpu.sync_copy(src_hbm.at[pl.ds(base + j * 128, 128)], sidx.at[j])
                pltpu.sync_copy(dst_hbm.at[pl.ds(base + j * 128, 128)], didx.at[j])
            for j in range(NSUB):
                for q in range(8):
                    v = sidx[j, pl.ds(q * 16, 16)] - half0
                    ok = (v >= 0) & (v < HALF)
                    clamp2[j, pl.ds(q * 16, 16)] = jnp.where(ok, v, DUMMY)
                pltpu.sync_copy(ones_v, table.at[clamp2.at[j]], add=True)
                for q in range(8):
                    v = didx[j, pl.ds(q * 16, 16)] - half0
                    ok = (v >= 0) & (v < HALF)
                    clamp2[j, pl.ds(q * 16, 16)] = jnp.where(ok, v, DUMMY)
                pltpu.sync_copy(ones_v, table.at[clamp2.at[j]], add=True)

        return 0

    lax.fori_loop(0, (NCHUNK + NTILE - 1) // NTILE, chunk_body, 0)
    plsc.subcore_barrier()
    local0 = s * (HALF // NTILE)
    pltpu.sync_copy(
        table.at[pl.ds(local0, HALF // NTILE)],
        deg_hbm.at[pl.ds(half0 + local0, HALF // NTILE)],
    )


@functools.partial(
    pl.kernel,
    out_type=jax.ShapeDtypeStruct((E, 16), jnp.float32),
    mesh=_mesh,
    compiler_params=_sc_params,
    scratch_types=[
        pltpu.VMEM((CH,), jnp.int32),
        pltpu.VMEM((CH,), jnp.int32),
        pltpu.VMEM((CH, 16), jnp.float32),
        pltpu.VMEM((CH, 16), jnp.float32),
        pltpu.SemaphoreType.DMA,
    ],
)
def _tsum_kernel(t_hbm, src_hbm, dst_hbm, ts_hbm, sidx, didx, rows_a, rows_b, sem):
    c = lax.axis_index("c")
    s = lax.axis_index("s")
    w = s * 2 + c

    def chunk_body(i, _):
        ch = w + 32 * i

        @pl.when(ch < NCHUNK)
        def _():
            base = ch * CH
            pltpu.sync_copy(src_hbm.at[pl.ds(base, CH)], sidx)
            pltpu.sync_copy(dst_hbm.at[pl.ds(base, CH)], didx)
            descs = []
            for j in range(NSUB):
                sl = pl.ds(j * 128, 128)
                descs.append(pltpu.async_copy(t_hbm.at[sidx.at[sl]], rows_a.at[sl], sem))
                descs.append(pltpu.async_copy(t_hbm.at[didx.at[sl]], rows_b.at[sl], sem))
            for d in descs:
                d.wait()

            def row_body(r, _):
                rows_a[r, :] = rows_a[r, :] + rows_b[r, :]
                return 0

            lax.fori_loop(0, CH, row_body, 0)
            pltpu.sync_copy(rows_a, ts_hbm.at[pl.ds(base, CH)])

        return 0

    lax.fori_loop(0, (NCHUNK + 31) // 32, chunk_body, 0)


@functools.lru_cache(maxsize=None)
def _make_acc(D):
    @functools.partial(
        pl.kernel,
        out_type=jax.ShapeDtypeStruct((MROWS, D), jnp.float32),
        mesh=_mesh,
        compiler_params=_sc_params,
        scratch_types=[
            pltpu.VMEM((CH, D), jnp.float32),
            pltpu.VMEM((NSUB, 128), jnp.int32),
            pltpu.VMEM((NSUB, 128), jnp.int32),
            pltpu.VMEM((NSUB, 128), jnp.int32),
            pltpu.VMEM((OUTG, 16), jnp.float32),
            pltpu.VMEM_SHARED((TBL, D), jnp.float32),
        ],
    )
    def _acc_kernel(xw_hbm, src_hbm, dst_hbm, deg_hbm, m_hbm,
                    rows_v, sidx, didx, clamp2, degb, table):
        c = lax.axis_index("c")
        s = lax.axis_index("s")
        half0 = c * HALF
        zvec = jnp.zeros((16,), jnp.float32)

        def zfill(i, _):
            for q in range(D // 16):
                rows_v[i, pl.ds(q * 16, 16)] = zvec
            return 0

        lax.fori_loop(0, 180, zfill, 0)

        def ztbl(t, _):
            pltpu.sync_copy(rows_v.at[pl.ds(0, 180)],
                            table.at[pl.ds(s * STRIPE + t * 180, 180)])
            return 0

        lax.fori_loop(0, STRIPE // 180, ztbl, 0)
        plsc.subcore_barrier()

        def chunk_body(i, _):
            ch = s + NTILE * i

            @pl.when(ch < NCHUNK)
            def _():
                base = ch * CH
                pltpu.sync_copy(xw_hbm.at[pl.ds(base, CH)], rows_v)
                for j in range(NSUB):
                    pltpu.sync_copy(src_hbm.at[pl.ds(base + j * 128, 128)], sidx.at[j])
                    pltpu.sync_copy(dst_hbm.at[pl.ds(base + j * 128, 128)], didx.at[j])
                for j in range(NSUB):
                    for q in range(8):
                        v = sidx[j, pl.ds(q * 16, 16)] - half0
                        ok = (v >= 0) & (v < HALF)
                        clamp2[j, pl.ds(q * 16, 16)] = jnp.where(ok, v, DUMMY)
                    pltpu.sync_copy(rows_v.at[pl.ds(j * 128, 128)],
                                    table.at[clamp2.at[j]], add=True)
                    for q in range(8):
                        v = didx[j, pl.ds(q * 16, 16)] - half0
                        ok = (v >= 0) & (v < HALF)
                        clamp2[j, pl.ds(q * 16, 16)] = jnp.where(ok, v, DUMMY)
                    pltpu.sync_copy(rows_v.at[pl.ds(j * 128, 128)],
                                    table.at[clamp2.at[j]], add=True)

            return 0

        lax.fori_loop(0, (NCHUNK + NTILE - 1) // NTILE, chunk_body, 0)
        plsc.subcore_barrier()

        rows_per_tile = HALF // NTILE
        local0 = s * rows_per_tile
        one = jnp.float32(1.0)
        zero = jnp.float32(0.0)

        def out_body(t, _):
            loc = local0 + t * OUTG
            pltpu.sync_copy(table.at[pl.ds(loc, OUTG)], rows_v.at[pl.ds(0, OUTG)])
            pltpu.sync_copy(deg_hbm.at[pl.ds(half0 + loc, OUTG)], degb)
            for r in range(OUTG):
                drow = degb[r, :]
                brow = jnp.where(drow >= 2.0, one / jnp.maximum(drow, one), zero)
                for q in range(D // 16):
                    rows_v[r, pl.ds(q * 16, 16)] = rows_v[r, pl.ds(q * 16, 16)] * brow
            pltpu.sync_copy(rows_v.at[pl.ds(0, OUTG)], m_hbm.at[pl.ds(half0 + loc, OUTG)])
            return 0

        lax.fori_loop(0, rows_per_tile // OUTG, out_body, 0)

    return _acc_kernel


@functools.partial(
    pl.kernel,
    out_type=jax.ShapeDtypeStruct((E, NHID), jnp.float32),
    mesh=_mesh,
    compiler_params=_sc_params,
    scratch_types=[
        pltpu.VMEM((CHC,), jnp.int32),
        pltpu.VMEM((CHC,), jnp.int32),
        pltpu.VMEM((CHC, NHID), jnp.float32),
        pltpu.VMEM((CHC, NHID), jnp.float32),
        pltpu.VMEM((CHC, 32), jnp.float32),
        pltpu.VMEM((CHC, 32), jnp.float32),
        pltpu.SemaphoreType.DMA,
    ],
)
def _comb64_kernel(m_hbm, src_hbm, dst_hbm, xwa_hbm, xwb_hbm, g_hbm,
                   sidx, didx, rows_a, rows_b, rows_ca, rows_cb, sem):
    c = lax.axis_index("c")
    s = lax.axis_index("s")
    w = s * 2 + c

    def chunk_body(i, _):
        ch = w + 32 * i

        @pl.when(ch < NCHUNKC)
        def _():
            base = ch * CHC
            pltpu.sync_copy(src_hbm.at[pl.ds(base, CHC)], sidx)
            pltpu.sync_copy(dst_hbm.at[pl.ds(base, CHC)], didx)
            descs = [pltpu.async_copy(xwa_hbm.at[pl.ds(base, CHC)], rows_ca, sem),
                     pltpu.async_copy(xwb_hbm.at[pl.ds(base, CHC)], rows_cb, sem)]
            for j in range(NSUBC):
                sl = pl.ds(j * 128, 128)
                descs.append(pltpu.async_copy(m_hbm.at[sidx.at[sl]], rows_a.at[sl], sem))
                descs.append(pltpu.async_copy(m_hbm.at[didx.at[sl]], rows_b.at[sl], sem))
            for d in descs:
                d.wait()

            def row_body(r, _):
                for q in range(4):
                    sl = pl.ds(q * 16, 16)
                    hsl = pl.ds((q % 2) * 16, 16)
                    half = rows_ca[r, hsl] if q < 2 else rows_cb[r, hsl]
                    rows_a[r, sl] = rows_a[r, sl] + rows_b[r, sl] + half
                return 0

            lax.fori_loop(0, CHC, row_body, 0)
            pltpu.sync_copy(rows_a, g_hbm.at[pl.ds(base, CHC)])

        return 0

    lax.fori_loop(0, (NCHUNKC + 31) // 32, chunk_body, 0)


@functools.partial(
    pl.kernel,
    out_type=jax.ShapeDtypeStruct((E, 16), jnp.float32),
    mesh=_mesh,
    compiler_params=_sc_params,
    scratch_types=[
        pltpu.VMEM((CHC,), jnp.int32),
        pltpu.VMEM((CHC,), jnp.int32),
        pltpu.VMEM((CHC, 16), jnp.float32),
        pltpu.VMEM((CHC, 16), jnp.float32),
        pltpu.VMEM((CHC, 16), jnp.float32),
        pltpu.SemaphoreType.DMA,
    ],
)
def _comb16_kernel(m_hbm, src_hbm, dst_hbm, xw_hbm, g_hbm,
                   sidx, didx, rows_a, rows_b, rows_c, sem):
    c = lax.axis_index("c")
    s = lax.axis_index("s")
    w = s * 2 + c

    def chunk_body(i, _):
        ch = w + 32 * i

        @pl.when(ch < NCHUNKC)
        def _():
            base = ch * CHC
            pltpu.sync_copy(src_hbm.at[pl.ds(base, CHC)], sidx)
            pltpu.sync_copy(dst_hbm.at[pl.ds(base, CHC)], didx)
            descs = [pltpu.async_copy(xw_hbm.at[pl.ds(base, CHC)], rows_c, sem)]
            for j in range(NSUBC):
                sl = pl.ds(j * 128, 128)
                descs.append(pltpu.async_copy(m_hbm.at[sidx.at[sl]], rows_a.at[sl], sem))
                descs.append(pltpu.async_copy(m_hbm.at[didx.at[sl]], rows_b.at[sl], sem))
            for d in descs:
                d.wait()

            def row_body(r, _):
                rows_a[r, :] = rows_a[r, :] + rows_b[r, :] + rows_c[r, :]
                return 0

            lax.fori_loop(0, CHC, row_body, 0)
            pltpu.sync_copy(rows_a, g_hbm.at[pl.ds(base, CHC)])

        return 0

    lax.fori_loop(0, (NCHUNKC + 31) // 32, chunk_body, 0)


def _row_block(d):
    return pl.BlockSpec((BK, d), lambda i: (i, 0))


def _full_block(shape):
    n = len(shape)
    return pl.BlockSpec(shape, lambda i: (0,) * n)


def _t16_tc(deg16):
    def body(d_ref, o_ref):
        o_ref[...] = jnp.where(d_ref[...] != 1.0, 1.0, 0.0)

    return pl.pallas_call(
        body,
        grid=(32,),
        in_specs=[pl.BlockSpec((MROWS // 32, 16), lambda i: (i, 0))],
        out_specs=pl.BlockSpec((MROWS // 32, 16), lambda i: (i, 0)),
        out_shape=jax.ShapeDtypeStruct((MROWS, 16), jnp.float32),
    )(deg16)


def _dinv_narrow(tsum):
    def body(t_ref, o_ref):
        o_ref[...] = 1.0 / (1.0 + t_ref[:, :1])

    return pl.pallas_call(
        body,
        grid=(GRID,),
        in_specs=[_row_block(16)],
        out_specs=_row_block(1),
        out_shape=jax.ShapeDtypeStruct((E, 1), jnp.float32),
    )(tsum)


def _dense_first(edge_attr, wa, wb):
    def body(a_ref, wa_ref, wb_ref, oa_ref, ob_ref):
        a = a_ref[...]
        oa_ref[...] = jnp.dot(a, wa_ref[...], preferred_element_type=jnp.float32)
        ob_ref[...] = jnp.dot(a, wb_ref[...], preferred_element_type=jnp.float32)

    return pl.pallas_call(
        body,
        grid=(GRID,),
        in_specs=[_row_block(D_EDGE), _full_block((D_EDGE, 32)), _full_block((D_EDGE, 32))],
        out_specs=[_row_block(32), _row_block(32)],
        out_shape=[jax.ShapeDtypeStruct((E, 32), jnp.float32),
                   jax.ShapeDtypeStruct((E, 32), jnp.float32)],
    )(edge_attr, wa, wb)


def _dense_mid(g, dinv2, b, wa, wb):
    def body(g_ref, d_ref, b_ref, wa_ref, wb_ref, oa_ref, ob_ref):
        h = jnp.maximum(g_ref[...] * d_ref[...] + b_ref[...], 0.0)
        oa_ref[...] = jnp.dot(h, wa_ref[...], preferred_element_type=jnp.float32)
        ob_ref[...] = jnp.dot(h, wb_ref[...], preferred_element_type=jnp.float32)

    return pl.pallas_call(
        body,
        grid=(GRID,),
        in_specs=[_row_block(NHID), _row_block(1), _full_block((1, NHID)),
                  _full_block((NHID, 32)), _full_block((NHID, 32))],
        out_specs=[_row_block(32), _row_block(32)],
        out_shape=[jax.ShapeDtypeStruct((E, 32), jnp.float32),
                   jax.ShapeDtypeStruct((E, 32), jnp.float32)],
    )(g, dinv2, b, wa, wb)


def _dense_mid16(g, dinv2, b, w):
    def body(g_ref, d_ref, b_ref, w_ref, o_ref):
        h = jnp.maximum(g_ref[...] * d_ref[...] + b_ref[...], 0.0)
        o_ref[...] = jnp.dot(h, w_ref[...], preferred_element_type=jnp.float32)

    return pl.pallas_call(
        body,
        grid=(GRID,),
        in_specs=[_row_block(NHID), _row_block(1), _full_block((1, NHID)),
                  _full_block((NHID, D_EDGE))],
        out_specs=_row_block(D_EDGE),
        out_shape=jax.ShapeDtypeStruct((E, D_EDGE), jnp.float32),
    )(g, dinv2, b, w)


def _dense_kv(g, dinv2, b, wka, wkb, wva, wvb):
    def body(g_ref, d_ref, b_ref, wka_ref, wkb_ref, wva_ref, wvb_ref,
             oka_ref, okb_ref, ova_ref, ovb_ref):
        h = jnp.maximum(g_ref[...] * d_ref[...] + b_ref[...], 0.0)
        oka_ref[...] = jnp.dot(h, wka_ref[...], preferred_element_type=jnp.float32)
        okb_ref[...] = jnp.dot(h, wkb_ref[...], preferred_element_type=jnp.float32)
        ova_ref[...] = jnp.dot(h, wva_ref[...], preferred_element_type=jnp.float32)
        ovb_ref[...] = jnp.dot(h, wvb_ref[...], preferred_element_type=jnp.float32)

    rb = [_row_block(32)] * 4
    sh = [jax.ShapeDtypeStruct((E, 32), jnp.float32)] * 4
    return pl.pallas_call(
        body,
        grid=(GRID,),
        in_specs=[_row_block(NHID), _row_block(1), _full_block((1, NHID))]
                 + [_full_block((NHID, 32))] * 4,
        out_specs=rb,
        out_shape=sh,
    )(g, dinv2, b, wka, wkb, wva, wvb)


def _tiny_q(spad, wq, bq):
    def body(s_ref, w_ref, b_ref, o_ref):
        o_ref[...] = jnp.dot(s_ref[...], w_ref[...],
                             preferred_element_type=jnp.float32) + b_ref[...]

    return pl.pallas_call(
        body,
        grid=(1,),
        in_specs=[_full_block((KPAD, NHID)), _full_block((NHID, NHID)), _full_block((1, NHID))],
        out_specs=_full_block((KPAD, NHID)),
        out_shape=jax.ShapeDtypeStruct((KPAD, NHID), jnp.float32),
    )(spad, wq, bq)


def _attn(gk, gv, dinv2, bk, bv, qt):
    scale = 1.0 / math.sqrt(NHID)

    def body(gk_ref, gv_ref, d_ref, bk_ref, bv_ref, qt_ref, a_ref, p_ref):
        i = pl.program_id(0)
        kmat = gk_ref[...] * d_ref[...] + bk_ref[...]
        vmat = gv_ref[...] * d_ref[...] + bv_ref[...]
        sc = jnp.dot(kmat, qt_ref[...], preferred_element_type=jnp.float32) * scale
        col = lax.broadcasted_iota(jnp.int32, (BK, KPAD), 1)
        sc = jnp.where(col < NUM_SEEDS, sc, -1e30)
        mx = jnp.max(sc, axis=1, keepdims=True)
        ex = jnp.exp(sc - mx)
        a = ex / jnp.sum(ex, axis=1, keepdims=True)
        a_ref[...] = a

        @pl.when(i == 0)
        def _():
            p_ref[...] = jnp.zeros_like(p_ref)

        p_ref[...] += lax.dot_general(a, vmat, (((0,), (0,)), ((), ())),
                                      preferred_element_type=jnp.float32)

    return pl.pallas_call(
        body,
        grid=(GRID,),
        in_specs=[_row_block(NHID), _row_block(NHID), _row_block(1),
                  _full_block((1, NHID)), _full_block((1, NHID)),
                  _full_block((NHID, KPAD))],
        out_specs=[_row_block(KPAD), _full_block((KPAD, NHID))],
        out_shape=[jax.ShapeDtypeStruct((E, KPAD), jnp.float32),
                   jax.ShapeDtypeStruct((KPAD, NHID), jnp.float32)],
    )(gk, gv, dinv2, bk, bv, qt)


def _tiny_post(p, q, ln0g, ln0b, wo, bo, ln1g, ln1b, wu0):
    eps = 1e-5

    def ln(xv, g_ref, b_ref):
        mu = jnp.mean(xv, axis=-1, keepdims=True)
        var = jnp.mean((xv - mu) ** 2, axis=-1, keepdims=True)
        return (xv - mu) / jnp.sqrt(var + eps) * g_ref[...] + b_ref[...]

    def body(p_ref, q_ref, g0_ref, b0_ref, wo_ref, bo_ref, g1_ref, b1_ref,
             wu_ref, o_ref):
        o = q_ref[...] + p_ref[...]
        o = ln(o, g0_ref, b0_ref)
        o = o + jnp.maximum(jnp.dot(o, wo_ref[...],
                                    preferred_element_type=jnp.float32) + bo_ref[...], 0.0)
        o = ln(o, g1_ref, b1_ref)
        o_ref[...] = jnp.dot(o, wu_ref[...], preferred_element_type=jnp.float32)

    return pl.pallas_call(
        body,
        grid=(1,),
        in_specs=[_full_block((KPAD, NHID)), _full_block((KPAD, NHID)),
                  _full_block((1, NHID)), _full_block((1, NHID)),
                  _full_block((NHID, NHID)), _full_block((1, NHID)),
                  _full_block((1, NHID)), _full_block((1, NHID)),
                  _full_block((NHID, NHID))],
        out_specs=_full_block((KPAD, NHID)),
        out_shape=jax.ShapeDtypeStruct((KPAD, NHID), jnp.float32),
    )(p, q, ln0g, ln0b, wo, bo, ln1g, ln1b, wu0)


def _dense_a(a, m):
    def body(a_ref, m_ref, oa_ref, ob_ref):
        x = a_ref[...]
        oa_ref[...] = jnp.dot(x, m_ref[...][:, :32], preferred_element_type=jnp.float32)
        ob_ref[...] = jnp.dot(x, m_ref[...][:, 32:], preferred_element_type=jnp.float32)

    return pl.pallas_call(
        body,
        grid=(GRID,),
        in_specs=[_row_block(KPAD), _full_block((KPAD, NHID))],
        out_specs=[_row_block(32), _row_block(32)],
        out_shape=[jax.ShapeDtypeStruct((E, 32), jnp.float32),
                   jax.ShapeDtypeStruct((E, 32), jnp.float32)],
    )(a, m)


def _dense_final(g, dinv2, b):
    def body(g_ref, d_ref, b_ref, o_ref):
        o_ref[...] = g_ref[...] * d_ref[...] + b_ref[...]

    return pl.pallas_call(
        body,
        grid=(GRID,),
        in_specs=[_row_block(D_EDGE), _row_block(1), _full_block((1, D_EDGE))],
        out_specs=_row_block(D_EDGE),
        out_shape=jax.ShapeDtypeStruct((E, D_EDGE), jnp.float32),
    )(g, dinv2, b)


def kernel(x, edge_index, edge_attr, batch, params):
    src = edge_index[0]
    dst = edge_index[1]

    deg16 = _deg_kernel(src, dst)
    t16 = _t16_tc(deg16)
    tsum = _tsum_kernel(t16, src, dst)
    dinv2 = _dinv_narrow(tsum)

    acc32 = _make_acc(32)
    acc16 = _make_acc(16)

    def hpipe64(xwa, xwb):
        ma = acc32(xwa, src, dst, deg16)
        mb = acc32(xwb, src, dst, deg16)
        m = jnp.concatenate([ma, mb], axis=1)
        return _comb64_kernel(m, src, dst, xwa, xwb)

    p = params
    b_ = lambda v: v.reshape(1, -1)
    halves = lambda w: (w[:, :32], w[:, 32:])

    w0a, w0b = halves(p["convs"][0]["W"])
    xw0a, xw0b = _dense_first(edge_attr, w0a, w0b)
    g0 = hpipe64(xw0a, xw0b)
    w1a, w1b = halves(p["convs"][1]["W"])
    xw1a, xw1b = _dense_mid(g0, dinv2, b_(p["convs"][0]["b"]), w1a, w1b)
    g1 = hpipe64(xw1a, xw1b)
    wka, wkb = halves(p["layer_k"]["W"])
    wva, wvb = halves(p["layer_v"]["W"])
    xwka, xwkb, xwva, xwvb = _dense_kv(g1, dinv2, b_(p["convs"][1]["b"]),
                                       wka, wkb, wva, wvb)
    gk = hpipe64(xwka, xwkb)
    gv = hpipe64(xwva, xwvb)

    spad = jnp.zeros((KPAD, NHID), jnp.float32).at[:NUM_SEEDS].set(p["S"][0])
    qpad = _tiny_q(spad, p["fc_q"]["W"], b_(p["fc_q"]["b"]))
    a, pacc = _attn(gk, gv, dinv2, b_(p["layer_k"]["b"]), b_(p["layer_v"]["b"]),
                    qpad.T)
    mmat = _tiny_post(pacc, qpad, b_(p["ln0"]["g"]), b_(p["ln0"]["b"]),
                      p["fc_o"]["W"], b_(p["fc_o"]["b"]),
                      b_(p["ln1"]["g"]), b_(p["ln1"]["b"]),
                      p["unconvs"][0]["W"])

    xw3a, xw3b = _dense_a(a, mmat)
    g3 = hpipe64(xw3a, xw3b)
    wu1a, wu1b = halves(p["unconvs"][1]["W"])
    xw4a, xw4b = _dense_mid(g3, dinv2, b_(p["unconvs"][0]["b"]), wu1a, wu1b)
    g4 = hpipe64(xw4a, xw4b)
    xw5 = _dense_mid16(g4, dinv2, b_(p["unconvs"][1]["b"]), p["last"]["W"])
    m5 = acc16(xw5, src, dst, deg16)
    g5 = _comb16_kernel(m5, src, dst, xw5)
    return _dense_final(g5, dinv2, b_(p["last"]["b"]))
